# baseline (speedup 1.0000x reference)
import functools

import jax
import jax.numpy as jnp
from jax import lax
from jax.experimental import pallas as pl
from jax.experimental.pallas import tpu as pltpu
from jax.experimental.pallas import tpu_sc as plsc

TEMP_INV = 10.0

NC = 2
NS = 16
NW = NC * NS
LANES = 16

B = 48
BT = 16


def _sc_scores(user_rep, item_rep, unodes, pnodes, neg_base):
    E = unodes.shape[0]
    D = user_rep.shape[1]
    num_items = item_rep.shape[0]
    EW = E // NW
    NBLK = (EW - BT) // B
    NPAIR = NBLK // 2
    assert NBLK == 2 * NPAIR and NBLK * B + BT == EW

    mesh = plsc.VectorSubcoreMesh(core_axis_name="c", subcore_axis_name="s")

    @functools.partial(
        pl.kernel,
        out_type=jax.ShapeDtypeStruct((E,), jnp.float32),
        mesh=mesh,
        compiler_params=pltpu.CompilerParams(
            needs_layout_passes=False, disable_bounds_checks=True),
        scratch_types=[
            pltpu.VMEM((EW,), jnp.int32),
            pltpu.VMEM((EW,), jnp.int32),
            pltpu.VMEM((EW,), jnp.int32),
            pltpu.VMEM((3 * B, D), jnp.float32),
            pltpu.VMEM((3 * B, D), jnp.float32),
            pltpu.VMEM((EW,), jnp.float32),
            pltpu.SemaphoreType.DMA,
            pltpu.SemaphoreType.DMA,
        ],
    )
    def scores_kernel(user_hbm, item_hbm, un_hbm, pn_hbm, negb_hbm, out_hbm,
                      uidx, pidx, nidx, r0, r1, scores, sem0, sem1):
        wid = lax.axis_index("s") * NC + lax.axis_index("c")
        wbase = wid * EW

        pltpu.sync_copy(un_hbm.at[pl.ds(wbase, EW)], uidx)
        pltpu.sync_copy(pn_hbm.at[pl.ds(wbase, EW)], pidx)
        pltpu.sync_copy(negb_hbm.at[pl.ds(wbase, EW)], nidx)

        def fix_body(i, carry):
            sl = pl.ds(i * LANES, LANES)
            nb = nidx[sl]
            pp = pidx[sl]
            bumped = jnp.where(nb + 1 == num_items, 0, nb + 1)
            nidx[sl] = jnp.where(nb == pp, bumped, nb)
            return carry

        lax.fori_loop(0, EW // LANES, fix_body, 0)

        def fire(j, rows, sem, n=B):
            sl = pl.ds(j * B, n)
            pltpu.async_copy(user_hbm.at[uidx.at[sl]], rows.at[pl.ds(0, n)], sem)
            pltpu.async_copy(item_hbm.at[pidx.at[sl]], rows.at[pl.ds(B, n)], sem)
            pltpu.async_copy(item_hbm.at[nidx.at[sl]], rows.at[pl.ds(2 * B, n)], sem)

        def wait_all(rows, sem):
            pltpu.make_async_copy(user_hbm.at[pl.ds(0, 3 * B)], rows, sem).wait()

        def wait_tail(rows, sem):
            for s in range(3):
                pltpu.make_async_copy(
                    user_hbm.at[pl.ds(0, BT)], rows.at[pl.ds(s * B, BT)], sem).wait()

        lane = jnp.arange(LANES, dtype=jnp.int32)

        def hsum(acc):
            for o in (8, 4, 2, 1):
                idx = (lane + o) & (LANES - 1)
                acc = acc + jnp.take_along_axis(acc, idx, axis=0)
            return acc

        def compute(j, rows, nedges=B):
            for g in range(nedges // LANES):
                vec = jnp.zeros((LANES,), jnp.float32)
                for i in range(LANES):
                    e = g * LANES + i
                    acc = jnp.zeros((LANES,), jnp.float32)
                    for k in range(D // LANES):
                        sl = pl.ds(k * LANES, LANES)
                        acc = acc + rows[e, sl] * (rows[B + e, sl] - rows[2 * B + e, sl])
                    vec = jnp.where(lane == i, hsum(acc), vec)
                scores[pl.ds(j * B + g * LANES, LANES)] = vec * TEMP_INV

        fire(0, r0, sem0)
        fire(1, r1, sem1)

        def pair_body(t, carry):
            j0 = 2 * t
            wait_all(r0, sem0)
            compute(j0, r0)

            @pl.when(t < NPAIR - 1)
            def _():
                fire(j0 + 2, r0, sem0)

            wait_all(r1, sem1)
            compute(j0 + 1, r1)

            @pl.when(t < NPAIR - 1)
            def _():
                fire(j0 + 3, r1, sem1)

            return carry

        lax.fori_loop(0, NPAIR, pair_body, 0)

        fire(NBLK, r0, sem0, n=BT)
        wait_tail(r0, sem0)
        compute(NBLK, r0, nedges=BT)

        pltpu.sync_copy(scores, out_hbm.at[pl.ds(wbase, EW)])

    return scores_kernel(user_rep, item_rep, unodes, pnodes, neg_base)


def _tc_loss(scores):
    E = scores.shape[0]
    s2d = scores.reshape(E // 128, 128)

    def loss_body(s_ref, o_ref):
        x = s_ref[...]
        sig = 1.0 / (1.0 + jnp.exp(-x))
        o_ref[0, 0] = -jnp.sum(jnp.log2(sig)) / E

    out = pl.pallas_call(
        loss_body,
        out_shape=jax.ShapeDtypeStruct((1, 1), jnp.float32),
        out_specs=pl.BlockSpec(memory_space=pltpu.SMEM),
    )(s2d)
    return out[0, 0]


def kernel(user_rep, item_rep, edge_index):
    E = edge_index.shape[1]
    num_items = item_rep.shape[0]
    neg_base = jax.random.randint(
        jax.random.key(42), (E,), 0, num_items, dtype=jnp.int32)
    scores = _sc_scores(user_rep, item_rep, edge_index[0], edge_index[1], neg_base)
    return _tc_loss(scores)
